# baseline (speedup 1.0000x reference)
import functools

import jax
import jax.numpy as jnp
from jax import lax
from jax.experimental import pallas as pl
from jax.experimental.pallas import tpu as pltpu
from jax.experimental.pallas import tpu_sc as plsc

P = 128
O = 16
NC, NS = 2, 16
NW = NC * NS
IDXW = 128


def _proj_body(e0_ref, e1_ref, w_ref, b_ref, t0_ref, t1_ref):
    dn = (((1,), (1,)), ((), ()))
    t0_ref[...] = lax.dot_general(
        e0_ref[...], w_ref[:, :P],
        dimension_numbers=dn,
        preferred_element_type=jnp.float32,
        precision=lax.Precision.HIGHEST,
    ) + b_ref[...]
    t1_ref[...] = lax.dot_general(
        e1_ref[...], w_ref[:, P:],
        dimension_numbers=dn,
        preferred_element_type=jnp.float32,
        precision=lax.Precision.HIGHEST,
    )


def _project_tables(E0, E1, W, b2d):
    V = E0.shape[0]
    return pl.pallas_call(
        _proj_body,
        out_shape=[
            jax.ShapeDtypeStruct((V, O), jnp.float32),
            jax.ShapeDtypeStruct((V, O), jnp.float32),
        ],
    )(E0, E1, W, b2d)


def _untile_body(r_ref, out_ref):
    chunks = [r_ref[:, pl.ds(16 * u, 16)] for u in range(8)]
    stacked = jnp.stack(chunks, axis=1)
    out_ref[...] = stacked.reshape(out_ref.shape)


def _untile(r, B):
    grid = 8
    rows = r.shape[0] // grid
    return pl.pallas_call(
        _untile_body,
        grid=(grid,),
        in_specs=[pl.BlockSpec((rows, 128), lambda i: (i, 0))],
        out_specs=pl.BlockSpec((rows * 8, O), lambda i: (i, 0)),
        out_shape=jax.ShapeDtypeStruct((B, O), jnp.float32),
    )(r)


@functools.lru_cache(maxsize=None)
def _make_sc_gather_add(B):
    bpw = B // NW
    nch = bpw // IDXW
    mesh = plsc.VectorSubcoreMesh(
        core_axis_name="c", subcore_axis_name="s",
        num_cores=NC, num_subcores=NS,
    )

    @functools.partial(
        pl.kernel,
        mesh=mesh,
        out_type=jax.ShapeDtypeStruct((B // IDXW, IDXW, O), jnp.float32),
        scratch_types=[
            pltpu.VMEM((nch, IDXW), jnp.int32),
            pltpu.VMEM((nch, IDXW), jnp.int32),
            pltpu.VMEM((bpw, O), jnp.float32),
            pltpu.VMEM((bpw, O), jnp.float32),
            pltpu.VMEM((bpw // IDXW, IDXW, O), jnp.float32),
            pltpu.SemaphoreType.DMA,
        ],
        compiler_params=pltpu.CompilerParams(use_tc_tiling_on_sc=False),
    )
    def sc_kernel(x0_hbm, x1_hbm, t0_hbm, t1_hbm, out_hbm,
                  i0_v, i1_v, r0_v, r1_v, of_v, sem):
        wid = lax.axis_index("s") * NC + lax.axis_index("c")
        rowbase = wid * nch
        pltpu.sync_copy(x0_hbm.at[pl.ds(rowbase, nch)], i0_v)
        pltpu.sync_copy(x1_hbm.at[pl.ds(rowbase, nch)], i1_v)
        copies = []
        for j in range(nch):
            dst = pl.ds(j * IDXW, IDXW)
            copies.append(pltpu.async_copy(
                t0_hbm.at[i0_v.at[j]], r0_v.at[dst], sem))
            copies.append(pltpu.async_copy(
                t1_hbm.at[i1_v.at[j]], r1_v.at[dst], sem))
        for c in copies:
            c.wait()

        @plsc.parallel_loop(0, bpw, 1, unroll=8)
        def add_row(i):
            of_v[i // IDXW, i % IDXW] = r0_v[i] + r1_v[i]

        blk = bpw // IDXW
        pltpu.sync_copy(of_v, out_hbm.at[pl.ds(wid * blk, blk)])

    return sc_kernel


def kernel(x, E0, E1, W, b):
    B = x.shape[0]
    assert B % (NW * IDXW) == 0
    t0, t1 = _project_tables(E0, E1, W, b.reshape(1, O))
    xi = x.astype(jnp.int32)
    x0 = xi[:, 0].reshape(B // IDXW, IDXW)
    x1 = xi[:, 1].reshape(B // IDXW, IDXW)
    r3 = _make_sc_gather_add(B)(x0, x1, t0, t1)
    return r3.reshape(B, O)


# speedup vs baseline: 1.1017x; 1.1017x over previous
import functools

import jax
import jax.numpy as jnp
from jax import lax
from jax.experimental import pallas as pl
from jax.experimental.pallas import tpu as pltpu
from jax.experimental.pallas import tpu_sc as plsc

P = 128
O = 16
NC, NS = 2, 16
NW = NC * NS
IDXW = 128


def _proj_body(e0_ref, e1_ref, w_ref, b_ref, t0_ref, t1_ref):
    dn = (((1,), (1,)), ((), ()))
    t0_ref[...] = lax.dot_general(
        e0_ref[...], w_ref[:, :P],
        dimension_numbers=dn,
        preferred_element_type=jnp.float32,
        precision=lax.Precision.HIGHEST,
    ) + b_ref[...]
    t1_ref[...] = lax.dot_general(
        e1_ref[...], w_ref[:, P:],
        dimension_numbers=dn,
        preferred_element_type=jnp.float32,
        precision=lax.Precision.HIGHEST,
    )


def _project_tables(E0, E1, W, b2d):
    V = E0.shape[0]
    return pl.pallas_call(
        _proj_body,
        out_shape=[
            jax.ShapeDtypeStruct((V, O), jnp.float32),
            jax.ShapeDtypeStruct((V, O), jnp.float32),
        ],
    )(E0, E1, W, b2d)


def _untile_body(r_ref, out_ref):
    chunks = [r_ref[:, pl.ds(16 * u, 16)] for u in range(8)]
    stacked = jnp.stack(chunks, axis=1)
    out_ref[...] = stacked.reshape(out_ref.shape)


def _untile(r, B):
    grid = 8
    rows = r.shape[0] // grid
    return pl.pallas_call(
        _untile_body,
        grid=(grid,),
        in_specs=[pl.BlockSpec((rows, 128), lambda i: (i, 0))],
        out_specs=pl.BlockSpec((rows * 8, O), lambda i: (i, 0)),
        out_shape=jax.ShapeDtypeStruct((B, O), jnp.float32),
    )(r)


@functools.lru_cache(maxsize=None)
def _make_sc_gather_add(B):
    bpw = B // NW
    nch = bpw // IDXW
    mesh = plsc.VectorSubcoreMesh(
        core_axis_name="c", subcore_axis_name="s",
        num_cores=NC, num_subcores=NS,
    )

    @functools.partial(
        pl.kernel,
        mesh=mesh,
        out_type=jax.ShapeDtypeStruct((O // 8 * B // IDXW * 8, IDXW), jnp.float32),
        scratch_types=[
            pltpu.VMEM((nch, IDXW), jnp.int32),
            pltpu.VMEM((nch, IDXW), jnp.int32),
            pltpu.VMEM((bpw, O), jnp.float32),
            pltpu.VMEM((bpw, O), jnp.float32),
            pltpu.VMEM((O // 8 * nch * 8, IDXW), jnp.float32),
            pltpu.SemaphoreType.DMA,
        ],
        compiler_params=pltpu.CompilerParams(
            use_tc_tiling_on_sc=False, needs_layout_passes=False),
    )
    def sc_kernel(x0_hbm, x1_hbm, t0_hbm, t1_hbm, out_hbm,
                  i0_v, i1_v, r0_v, r1_v, of_v, sem):
        wid = lax.axis_index("s") * NC + lax.axis_index("c")
        rowbase = wid * nch
        pltpu.sync_copy(x0_hbm.at[pl.ds(rowbase, nch)], i0_v)
        pltpu.sync_copy(x1_hbm.at[pl.ds(rowbase, nch)], i1_v)
        copies = []
        for j in range(nch):
            dst = pl.ds(j * IDXW, IDXW)
            copies.append(pltpu.async_copy(
                t0_hbm.at[i0_v.at[j]], r0_v.at[dst], sem))
            copies.append(pltpu.async_copy(
                t1_hbm.at[i1_v.at[j]], r1_v.at[dst], sem))
        for c in copies:
            c.wait()

        lanes = lax.iota(jnp.int32, 16)
        jfull = [jnp.zeros((16,), jnp.int32) + j for j in range(O)]
        for t in range(bpw // 16):
            rows = lanes + t * 16
            rc, w = t // 8, t % 8
            for j in range(O):
                a = plsc.load_gather(r0_v, [rows, jfull[j]])
                b = plsc.load_gather(r1_v, [rows, jfull[j]])
                row = (j // 8) * (nch * 8) + (j % 8) + rc * 8
                of_v[row, pl.ds(w * 16, 16)] = a + b

        blk = nch * 8
        for g in range(O // 8):
            pltpu.sync_copy(of_v.at[pl.ds(g * blk, blk)],
                            out_hbm.at[pl.ds((g * NW + wid) * blk, blk)])

    return sc_kernel


def kernel(x, E0, E1, W, b):
    B = x.shape[0]
    assert B % (NW * IDXW) == 0
    t0, t1 = _project_tables(E0, E1, W, b.reshape(1, O))
    xi = x.astype(jnp.int32)
    x0 = xi[:, 0].reshape(B // IDXW, IDXW)
    x1 = xi[:, 1].reshape(B // IDXW, IDXW)
    r2 = _make_sc_gather_add(B)(x0, x1, t0, t1)
    r4 = r2.reshape(O // 8, B // IDXW, 8, IDXW)
    return r4.transpose(1, 3, 0, 2).reshape(B, O)


# speedup vs baseline: 1.3447x; 1.2206x over previous
import functools

import jax
import jax.numpy as jnp
from jax import lax
from jax.experimental import pallas as pl
from jax.experimental.pallas import tpu as pltpu
from jax.experimental.pallas import tpu_sc as plsc

P = 128
O = 16
NC, NS = 2, 16
NW = NC * NS
IDXW = 128


def _proj_body(e0_ref, e1_ref, w_ref, b_ref, t0_ref, t1_ref):
    dn = (((1,), (1,)), ((), ()))
    t0_ref[...] = lax.dot_general(
        e0_ref[...], w_ref[:, :P],
        dimension_numbers=dn,
        preferred_element_type=jnp.float32,
        precision=lax.Precision.HIGHEST,
    ) + b_ref[...]
    t1_ref[...] = lax.dot_general(
        e1_ref[...], w_ref[:, P:],
        dimension_numbers=dn,
        preferred_element_type=jnp.float32,
        precision=lax.Precision.HIGHEST,
    )


def _project_tables(E0, E1, W, b2d):
    V = E0.shape[0]
    return pl.pallas_call(
        _proj_body,
        out_shape=[
            jax.ShapeDtypeStruct((V, O), jnp.float32),
            jax.ShapeDtypeStruct((V, O), jnp.float32),
        ],
    )(E0, E1, W, b2d)


def _untile_body(r_ref, out_ref):
    chunks = [r_ref[:, pl.ds(16 * u, 16)] for u in range(8)]
    stacked = jnp.stack(chunks, axis=1)
    out_ref[...] = stacked.reshape(out_ref.shape)


def _untile(r, B):
    grid = 8
    rows = r.shape[0] // grid
    return pl.pallas_call(
        _untile_body,
        grid=(grid,),
        in_specs=[pl.BlockSpec((rows, 128), lambda i: (i, 0))],
        out_specs=pl.BlockSpec((rows * 8, O), lambda i: (i, 0)),
        out_shape=jax.ShapeDtypeStruct((B, O), jnp.float32),
    )(r)


@functools.lru_cache(maxsize=None)
def _make_sc_gather_add(B):
    bpw = B // NW
    nch = bpw // IDXW
    mesh = plsc.VectorSubcoreMesh(
        core_axis_name="c", subcore_axis_name="s",
        num_cores=NC, num_subcores=NS,
    )

    @functools.partial(
        pl.kernel,
        mesh=mesh,
        out_type=jax.ShapeDtypeStruct((O // 8 * B // IDXW * 8, IDXW), jnp.float32),
        scratch_types=[
            pltpu.VMEM((nch, IDXW), jnp.int32),
            pltpu.VMEM((nch, IDXW), jnp.int32),
            pltpu.VMEM((bpw, O), jnp.float32),
            pltpu.VMEM((bpw, O), jnp.float32),
            pltpu.VMEM((O // 8 * nch * 8, IDXW), jnp.float32),
            pltpu.SemaphoreType.DMA,
        ],
        compiler_params=pltpu.CompilerParams(
            use_tc_tiling_on_sc=False, needs_layout_passes=False),
    )
    def sc_kernel(x0_hbm, x1_hbm, t0_hbm, t1_hbm, out_hbm,
                  i0_v, i1_v, r0_v, r1_v, of_v, sem):
        wid = lax.axis_index("s") * NC + lax.axis_index("c")
        rowbase = wid * nch
        pltpu.sync_copy(x0_hbm.at[pl.ds(rowbase, nch)], i0_v)
        pltpu.sync_copy(x1_hbm.at[pl.ds(rowbase, nch)], i1_v)
        copies = []
        for j in range(nch):
            dst = pl.ds(j * IDXW, IDXW)
            copies.append(pltpu.async_copy(
                t0_hbm.at[i0_v.at[j]], r0_v.at[dst], sem))
            copies.append(pltpu.async_copy(
                t1_hbm.at[i1_v.at[j]], r1_v.at[dst], sem))
        for c in copies:
            c.wait()

        @plsc.parallel_loop(0, bpw, 1, unroll=8)
        def add_row(i):
            r0_v[i] = r0_v[i] + r1_v[i]

        lanes = lax.iota(jnp.int32, 16)
        jfull = [jnp.zeros((16,), jnp.int32) + j for j in range(O)]

        @plsc.parallel_loop(0, bpw // 16, 1, unroll=2)
        def asm_blk(t):
            rows = lanes + t * 16
            rc, w = t // 8, t % 8
            for j in range(O):
                g = plsc.load_gather(r0_v, [rows, jfull[j]])
                row = (j // 8) * (nch * 8) + (j % 8) + rc * 8
                of_v[row, pl.ds(w * 16, 16)] = g

        blk = nch * 8
        for g in range(O // 8):
            pltpu.sync_copy(of_v.at[pl.ds(g * blk, blk)],
                            out_hbm.at[pl.ds((g * NW + wid) * blk, blk)])

    return sc_kernel


def kernel(x, E0, E1, W, b):
    B = x.shape[0]
    assert B % (NW * IDXW) == 0
    t0, t1 = _project_tables(E0, E1, W, b.reshape(1, O))
    xi = x.astype(jnp.int32)
    x0 = xi[:, 0].reshape(B // IDXW, IDXW)
    x1 = xi[:, 1].reshape(B // IDXW, IDXW)
    r2 = _make_sc_gather_add(B)(x0, x1, t0, t1)
    r4 = r2.reshape(O // 8, B // IDXW, 8, IDXW)
    return r4.transpose(1, 3, 0, 2).reshape(B, O)


# speedup vs baseline: 1.5327x; 1.1398x over previous
import functools

import jax
import jax.numpy as jnp
from jax import lax
from jax.experimental import pallas as pl
from jax.experimental.pallas import tpu as pltpu
from jax.experimental.pallas import tpu_sc as plsc

P = 128
O = 16
NC, NS = 2, 16
NW = NC * NS
IDXW = 128


def _proj_body(e0_ref, e1_ref, w_ref, b_ref, t0_ref, t1_ref):
    dn = (((1,), (1,)), ((), ()))
    vr = e0_ref.shape[0]
    for u in range(8):
        d0 = lax.dot_general(
            e0_ref[:, u, :], w_ref[:, :P], dimension_numbers=dn,
            preferred_element_type=jnp.float32,
            precision=lax.Precision.HIGHEST,
        ) + b_ref[...]
        t0_ref[:vr, pl.ds(u * O, O)] = d0
        t1_ref[:vr, pl.ds(u * O, O)] = lax.dot_general(
            e1_ref[:, u, :], w_ref[:, P:], dimension_numbers=dn,
            preferred_element_type=jnp.float32,
            precision=lax.Precision.HIGHEST,
        )


def _project_tables(E0r, E1r, W, b2d):
    return pl.pallas_call(
        _proj_body,
        out_shape=[
            jax.ShapeDtypeStruct((32, 128), jnp.float32),
            jax.ShapeDtypeStruct((32, 128), jnp.float32),
        ],
    )(E0r, E1r, W, b2d)


def _untile_body(r_ref, out_ref):
    chunks = [r_ref[:, pl.ds(16 * u, 16)] for u in range(8)]
    stacked = jnp.stack(chunks, axis=1)
    out_ref[...] = stacked.reshape(out_ref.shape)


def _untile(r, B):
    grid = 8
    rows = r.shape[0] // grid
    return pl.pallas_call(
        _untile_body,
        grid=(grid,),
        in_specs=[pl.BlockSpec((rows, 128), lambda i: (i, 0))],
        out_specs=pl.BlockSpec((rows * 8, O), lambda i: (i, 0)),
        out_shape=jax.ShapeDtypeStruct((B, O), jnp.float32),
    )(r)


@functools.lru_cache(maxsize=None)
def _make_sc_gather_add(B):
    bpw = B // NW
    nch = bpw // IDXW
    mesh = plsc.VectorSubcoreMesh(
        core_axis_name="c", subcore_axis_name="s",
        num_cores=NC, num_subcores=NS,
    )

    @functools.partial(
        pl.kernel,
        mesh=mesh,
        out_type=jax.ShapeDtypeStruct((O // 8 * B // IDXW * 8, IDXW), jnp.float32),
        scratch_types=[
            pltpu.VMEM((2 * nch, IDXW), jnp.int32),
            pltpu.VMEM((bpw, O), jnp.float32),
            pltpu.VMEM((bpw, O), jnp.float32),
            pltpu.VMEM((O // 8 * nch * 8, IDXW), jnp.float32),
            pltpu.SemaphoreType.DMA,
        ],
        compiler_params=pltpu.CompilerParams(
            use_tc_tiling_on_sc=False, needs_layout_passes=False),
    )
    def sc_kernel(xv_hbm, t0_hbm, t1_hbm, out_hbm,
                  ix_v, r0_v, r1_v, of_v, sem):
        wid = lax.axis_index("s") * NC + lax.axis_index("c")
        pltpu.sync_copy(xv_hbm.at[pl.ds(wid * 2 * nch, 2 * nch)], ix_v)
        copies = []
        for j in range(nch):
            dst = pl.ds(j * IDXW, IDXW)
            copies.append(pltpu.async_copy(
                t0_hbm.at[ix_v.at[2 * j]], r0_v.at[dst], sem))
            copies.append(pltpu.async_copy(
                t1_hbm.at[ix_v.at[2 * j + 1]], r1_v.at[dst], sem))
        for c in copies:
            c.wait()

        @plsc.parallel_loop(0, bpw, 1, unroll=8)
        def add_row(i):
            r0_v[i] = r0_v[i] + r1_v[i]

        lanes = lax.iota(jnp.int32, 16)
        jfull = [jnp.zeros((16,), jnp.int32) + j for j in range(O)]

        @plsc.parallel_loop(0, bpw // 16, 1, unroll=2)
        def asm_blk(t):
            rows = lanes + t * 16
            rc, w = t // 8, t % 8
            for j in range(O):
                g = plsc.load_gather(r0_v, [rows, jfull[j]])
                row = (j // 8) * (nch * 8) + (j % 8) + rc * 8
                of_v[row, pl.ds(w * 16, 16)] = g

        blk = nch * 8
        for g in range(O // 8):
            pltpu.sync_copy(of_v.at[pl.ds(g * blk, blk)],
                            out_hbm.at[pl.ds((g * NW + wid) * blk, blk)])

    return sc_kernel


def kernel(x, E0, E1, W, b):
    B = x.shape[0]
    assert B % (NW * IDXW) == 0
    t0p, t1p = _project_tables(
        E0.reshape(25, 8, P), E1.reshape(25, 8, P), W, b.reshape(1, O))
    t0 = t0p.reshape(256, O)
    t1 = t1p.reshape(256, O)
    xi = x.astype(jnp.int32)
    xv = xi.reshape(B // IDXW, IDXW, 2).transpose(0, 2, 1).reshape(
        2 * B // IDXW, IDXW)
    r2 = _make_sc_gather_add(B)(xv, t0, t1)
    r4 = r2.reshape(O // 8, B // IDXW, 8, IDXW)
    return r4.transpose(1, 3, 0, 2).reshape(B, O)
